# Initial kernel scaffold; baseline (speedup 1.0000x reference)
import jax
import jax.numpy as jnp
from jax.experimental import pallas as pl


def kernel(unknown, known, unknow_feats, known_feats):
    raise NotImplementedError("write your pallas kernel here")


# speedup vs baseline: 8.7746x; 8.7746x over previous
import functools

import jax
import jax.numpy as jnp
from jax import lax
from jax.experimental import pallas as pl
from jax.experimental.pallas import tpu as pltpu
from jax.experimental.pallas import tpu_sc as plsc

B = 4
N = 8192
M = 2048
CK = 768
CU = 384
NBLK = 256

NC = 2
NS = 16
NW = NC * NS
L = 16

CHUNKS_PER_BATCH = NW // B
C_PER_TILE = CK // CHUNKS_PER_BATCH
U_PER_TILE = CU // CHUNKS_PER_BATCH


def _knn_body(known_ref, unknown_ref, idx_ref, w_ref):
    k = known_ref[0]
    u = unknown_ref[0]
    mm = lax.dot_general(k, u, (((1,), (1,)), ((), ())),
                         precision=lax.Precision.HIGHEST,
                         preferred_element_type=jnp.float32)
    kn2 = jnp.sum(k * k, axis=1, keepdims=True)
    un2 = jnp.sum(u * u, axis=1)[None, :]
    d2 = kn2 - 2.0 * mm + un2
    iot = lax.broadcasted_iota(jnp.int32, d2.shape, 0)
    recips = []
    for t in range(3):
        mv = jnp.min(d2, axis=0, keepdims=True)
        sel = jnp.where(d2 == mv, iot, M)
        mi = jnp.min(sel, axis=0, keepdims=True)
        idx_ref[0, t, :] = mi[0]
        d2 = jnp.where(iot == mi, jnp.float32(jnp.inf), d2)
        dist = jnp.sqrt(jnp.maximum(mv, 0.0))
        recips.append(1.0 / (dist + 1e-8))
    norm = recips[0] + recips[1] + recips[2]
    for t in range(3):
        w_ref[0, t, :] = (recips[t] / norm)[0]


def _three_nn(unknown, known):
    return pl.pallas_call(
        _knn_body,
        grid=(B, N // NBLK),
        in_specs=[
            pl.BlockSpec((1, M, 3), lambda i, j: (i, 0, 0)),
            pl.BlockSpec((1, NBLK, 3), lambda i, j: (i, j, 0)),
        ],
        out_specs=[
            pl.BlockSpec((1, 3, NBLK), lambda i, j: (i, 0, j)),
            pl.BlockSpec((1, 3, NBLK), lambda i, j: (i, 0, j)),
        ],
        out_shape=[
            jax.ShapeDtypeStruct((B, 3, N), jnp.int32),
            jax.ShapeDtypeStruct((B, 3, N), jnp.float32),
        ],
    )(known, unknown)


def _interp_body(kf, idxh, wh, uf, out, idx_v, w_v, tab_v, row_v):
    cax = lax.axis_index("c")
    sax = lax.axis_index("s")
    wid = sax * NC + cax
    bi = wid // CHUNKS_PER_BATCH
    ci = lax.rem(wid, CHUNKS_PER_BATCH)

    pltpu.sync_copy(idxh.at[bi], idx_v)
    pltpu.sync_copy(wh.at[bi], w_v)

    def copy_body(r, carry):
        row = ci * U_PER_TILE + r
        pltpu.sync_copy(uf.at[bi, row], row_v)
        pltpu.sync_copy(row_v, out.at[bi, CK + row])
        return carry

    lax.fori_loop(0, U_PER_TILE, copy_body, 0)

    def chan_body(c, carry):
        ch = ci * C_PER_TILE + c
        pltpu.sync_copy(kf.at[bi, ch], tab_v)

        def vec_body(j, carry2):
            off = pl.multiple_of(j * L, L)
            acc = jnp.zeros((L,), jnp.float32)
            for t in range(3):
                ii = idx_v[t, pl.ds(off, L)]
                ww = w_v[t, pl.ds(off, L)]
                g = plsc.load_gather(tab_v, [ii])
                acc = acc + g * ww
            row_v[pl.ds(off, L)] = acc
            return carry2

        lax.fori_loop(0, N // L, vec_body, 0)
        pltpu.sync_copy(row_v, out.at[bi, ch])
        return carry

    lax.fori_loop(0, C_PER_TILE, chan_body, 0)


def _interpolate(kf, idx, w, uf):
    mesh = plsc.VectorSubcoreMesh(core_axis_name="c", subcore_axis_name="s")
    fn = functools.partial(
        pl.kernel,
        out_type=jax.ShapeDtypeStruct((B, CK + CU, N), jnp.float32),
        mesh=mesh,
        scratch_types=[
            pltpu.VMEM((3, N), jnp.int32),
            pltpu.VMEM((3, N), jnp.float32),
            pltpu.VMEM((M,), jnp.float32),
            pltpu.VMEM((N,), jnp.float32),
        ],
        compiler_params=pltpu.CompilerParams(needs_layout_passes=False),
    )(_interp_body)
    return fn(kf, idx, w, uf)


def kernel(unknown, known, unknow_feats, known_feats):
    idx, w = _three_nn(unknown, known)
    kf = known_feats.reshape(B, CK, M)
    uf = unknow_feats.reshape(B, CU, N)
    out = _interpolate(kf, idx, w, uf)
    return out.reshape(B, (CK + CU) // 3, 3, N)
